# baseline (speedup 1.0000x reference)
import functools

import jax
import jax.numpy as jnp
from jax import lax
from jax.experimental import pallas as pl
from jax.experimental.pallas import tpu as pltpu
from jax.experimental.pallas import tpu_sc as plsc

_D = 64
_DP = 128
_NB = 4096 * 200
_NC, _NS = 2, 16
_NW = _NC * _NS
_BPW = _NB // _NW
_C = 128
_NBUF = 2
_NCHUNK = _BPW // _C
_NROUNDS = _NCHUNK // _NBUF

_mesh = plsc.VectorSubcoreMesh(core_axis_name="c", subcore_axis_name="s")


@functools.partial(
    pl.kernel,
    out_type=jax.ShapeDtypeStruct((_NB, _D), jnp.float32),
    mesh=_mesh,
    scratch_types=[
        pltpu.VMEM((_BPW,), jnp.int32),
        pltpu.VMEM((_NBUF, _C, _DP), jnp.float32),
        pltpu.VMEM((_NBUF, _C, _D), jnp.float32),
        pltpu.SemaphoreType.DMA((_NBUF,)),
        pltpu.SemaphoreType.DMA((_NBUF,)),
    ],
)
def _gather_kernel(idx_hbm, table_hbm, out_hbm, idx_v, rows128, rows64,
                   sem_g, sem_s):
    wid = lax.axis_index("s") * _NC + lax.axis_index("c")
    base = wid * _BPW
    pltpu.sync_copy(idx_hbm.at[pl.ds(base, _BPW)], idx_v)

    def g_desc(c, b):
        return pltpu.make_async_copy(
            table_hbm.at[idx_v.at[pl.ds(c * _C, _C)]], rows128.at[b],
            sem_g.at[b])

    def s_desc(c, b):
        return pltpu.make_async_copy(
            rows64.at[b], out_hbm.at[pl.ds(base + c * _C, _C)], sem_s.at[b])

    def compact(b):
        def r_body(r8, carry):
            for r0 in range(8):
                r = r8 * 8 + r0
                for g in range(_D // 16):
                    rows64[b, r, pl.ds(16 * g, 16)] = (
                        rows128[b, r, pl.ds(16 * g, 16)])
            return carry

        lax.fori_loop(0, _C // 8, r_body, 0)

    for b in range(_NBUF):
        g_desc(b, b).start()

    def round_body(r, carry):
        c0 = r * _NBUF
        for b in range(_NBUF):
            g_desc(c0 + b, b).wait()
            compact(b)
            s_desc(c0 + b, b).start()
        for b in range(_NBUF):
            s_desc(c0 + b, b).wait()
            g_desc(c0 + _NBUF + b, b).start()
        return carry

    lax.fori_loop(0, _NROUNDS - 1, round_body, 0)

    c0 = (_NROUNDS - 1) * _NBUF
    for b in range(_NBUF):
        g_desc(c0 + b, b).wait()
        compact(b)
        s_desc(c0 + b, b).start()
    for b in range(_NBUF):
        s_desc(c0 + b, b).wait()


def kernel(x, table):
    idx = x.reshape(-1)
    tp = jnp.pad(table, ((0, 0), (0, _DP - _D)))
    out = _gather_kernel(idx, tp)
    return out.reshape(x.shape + (table.shape[1],))


# speedup vs baseline: 1.0093x; 1.0093x over previous
import functools

import jax
import jax.numpy as jnp
from jax import lax
from jax.experimental import pallas as pl
from jax.experimental.pallas import tpu as pltpu
from jax.experimental.pallas import tpu_sc as plsc

_D = 64
_DP = 128
_NB = 4096 * 200
_NC, _NS = 2, 16
_NW = _NC * _NS
_BPW = _NB // _NW
_C = 160
_NBUF = 2
_NCHUNK = _BPW // _C
_NROUNDS = _NCHUNK // _NBUF

_mesh = plsc.VectorSubcoreMesh(core_axis_name="c", subcore_axis_name="s")


@functools.partial(
    pl.kernel,
    out_type=jax.ShapeDtypeStruct((_NB, _D), jnp.float32),
    mesh=_mesh,
    scratch_types=[
        pltpu.VMEM((_BPW,), jnp.int32),
        pltpu.VMEM((_NBUF, _C, _DP), jnp.float32),
        pltpu.VMEM((_NBUF, _C, _D), jnp.float32),
        pltpu.SemaphoreType.DMA((_NBUF,)),
        pltpu.SemaphoreType.DMA((_NBUF,)),
    ],
)
def _gather_kernel(idx_hbm, table_hbm, out_hbm, idx_v, rows128, rows64,
                   sem_g, sem_s):
    wid = lax.axis_index("s") * _NC + lax.axis_index("c")
    base = wid * _BPW
    pltpu.sync_copy(idx_hbm.at[pl.ds(base, _BPW)], idx_v)

    def g_desc(c, b):
        return pltpu.make_async_copy(
            table_hbm.at[idx_v.at[pl.ds(c * _C, _C)]], rows128.at[b],
            sem_g.at[b])

    def s_desc(c, b):
        return pltpu.make_async_copy(
            rows64.at[b], out_hbm.at[pl.ds(base + c * _C, _C)], sem_s.at[b])

    def compact(b):
        def r_body(r8, carry):
            for r0 in range(8):
                r = r8 * 8 + r0
                for g in range(_D // 16):
                    rows64[b, r, pl.ds(16 * g, 16)] = (
                        rows128[b, r, pl.ds(16 * g, 16)])
            return carry

        lax.fori_loop(0, _C // 8, r_body, 0)

    for b in range(_NBUF):
        g_desc(b, b).start()

    def round_body(r, carry):
        c0 = r * _NBUF
        for b in range(_NBUF):
            g_desc(c0 + b, b).wait()
            compact(b)
            s_desc(c0 + b, b).start()
        for b in range(_NBUF):
            s_desc(c0 + b, b).wait()
            g_desc(c0 + _NBUF + b, b).start()
        return carry

    lax.fori_loop(0, _NROUNDS - 1, round_body, 0)

    c0 = (_NROUNDS - 1) * _NBUF
    for b in range(_NBUF):
        g_desc(c0 + b, b).wait()
        compact(b)
        s_desc(c0 + b, b).start()
    for b in range(_NBUF):
        s_desc(c0 + b, b).wait()


def kernel(x, table):
    idx = x.reshape(-1)
    tp = jnp.pad(table, ((0, 0), (0, _DP - _D)))
    out = _gather_kernel(idx, tp)
    return out.reshape(x.shape + (table.shape[1],))
